# baseline (speedup 1.0000x reference)
import jax
import jax.numpy as jnp
from jax import lax
from jax.experimental import pallas as pl

_RATIO = 0.7
_LMBDA = 0.5
_B = 16384
_C = 1000
_K = int(_B * _RATIO)

_BN = 512
_NB = _B // _BN


def _phase1_body(yh_ref, y_ref, nll_ref, tl_ref):
    yh = yh_ref[...]
    yy = y_ref[...]
    m = jnp.max(yy, axis=0, keepdims=True)
    ii = lax.broadcasted_iota(jnp.int32, yy.shape, 0)
    idx = jnp.min(jnp.where(yy == m, ii, _C), axis=0, keepdims=True)
    nll = -jnp.sum(jnp.where(ii == idx, yh, 0.0), axis=0, keepdims=True)
    dot = jnp.sum(yh * yy, axis=0, keepdims=True)
    nll_ref[...] = nll
    tl_ref[...] = nll + _LMBDA * (1.0 - dot)


def _phase2_body(nll_ref, tl_ref, out_ref):
    nll = nll_ref[...]
    tl = tl_ref[...]
    i32 = lax.bitcast_convert_type(tl, jnp.int32)
    keyi = jnp.where(i32 < 0, jnp.bitwise_not(i32),
                     jnp.bitwise_or(i32, jnp.int32(-(2**31))))
    u = lax.bitcast_convert_type(keyi, jnp.uint32)
    t = jnp.uint32(0)
    for b in range(31, -1, -1):
        cand = t | jnp.uint32(1 << b)
        cnt = jnp.sum((u >= cand).astype(jnp.int32))
        t = jnp.where(cnt >= _K, cand, t)
    gt = u > t
    eq = u == t
    cnt_gt = jnp.sum(gt.astype(jnp.int32))
    sum_gt = jnp.sum(jnp.where(gt, nll, 0.0))
    cnt_eq = jnp.sum(eq.astype(jnp.int32))
    sum_eq = jnp.sum(jnp.where(eq, nll, 0.0))
    rem = (_K - cnt_gt).astype(jnp.float32)
    total = sum_gt + rem * sum_eq / jnp.maximum(cnt_eq, 1).astype(jnp.float32)
    out_ref[...] = jnp.broadcast_to(total / jnp.float32(_K), (1, 1))


def kernel(y_hat, y):
    yht = y_hat.T
    yt = y.T
    nll, tl = pl.pallas_call(
        _phase1_body,
        grid=(_NB,),
        in_specs=[
            pl.BlockSpec((_C, _BN), lambda i: (0, i)),
            pl.BlockSpec((_C, _BN), lambda i: (0, i)),
        ],
        out_specs=[
            pl.BlockSpec((1, _BN), lambda i: (0, i)),
            pl.BlockSpec((1, _BN), lambda i: (0, i)),
        ],
        out_shape=[
            jax.ShapeDtypeStruct((1, _B), jnp.float32),
            jax.ShapeDtypeStruct((1, _B), jnp.float32),
        ],
    )(yht, yt)

    nll2 = nll.reshape(128, 128)
    tl2 = tl.reshape(128, 128)
    out = pl.pallas_call(
        _phase2_body,
        out_shape=jax.ShapeDtypeStruct((1, 1), jnp.float32),
    )(nll2, tl2)
    return out[0, 0]


# speedup vs baseline: 1.1728x; 1.1728x over previous
import jax
import jax.numpy as jnp
from jax import lax
from jax.experimental import pallas as pl

_RATIO = 0.7
_LMBDA = 0.5
_B = 16384
_C = 1000
_K = int(_B * _RATIO)

_BN = 2048
_NB = _B // _BN


def _phase1_body(yh_ref, y_ref, nll_ref, tl_ref):
    yh = yh_ref[...]
    yy = y_ref[...]
    m = jnp.max(yy, axis=0, keepdims=True)
    ii = lax.broadcasted_iota(jnp.int32, yy.shape, 0)
    idx = jnp.min(jnp.where(yy == m, ii, _C), axis=0, keepdims=True)
    nll = -jnp.sum(jnp.where(ii == idx, yh, 0.0), axis=0, keepdims=True)
    dot = jnp.sum(yh * yy, axis=0, keepdims=True)
    nll_ref[...] = nll
    tl_ref[...] = nll + _LMBDA * (1.0 - dot)


def _phase2_body(nll_ref, tl_ref, out_ref):
    nll = nll_ref[...]
    tl = tl_ref[...]
    i32 = lax.bitcast_convert_type(tl, jnp.int32)
    keyi = jnp.where(i32 < 0, jnp.bitwise_not(i32),
                     jnp.bitwise_or(i32, jnp.int32(-(2**31))))
    u = lax.bitcast_convert_type(keyi, jnp.uint32)
    t = jnp.uint32(0)
    for b in range(31, -1, -1):
        cand = t | jnp.uint32(1 << b)
        cnt = jnp.sum((u >= cand).astype(jnp.int32))
        t = jnp.where(cnt >= _K, cand, t)
    gt = u > t
    eq = u == t
    cnt_gt = jnp.sum(gt.astype(jnp.int32))
    sum_gt = jnp.sum(jnp.where(gt, nll, 0.0))
    cnt_eq = jnp.sum(eq.astype(jnp.int32))
    sum_eq = jnp.sum(jnp.where(eq, nll, 0.0))
    rem = (_K - cnt_gt).astype(jnp.float32)
    total = sum_gt + rem * sum_eq / jnp.maximum(cnt_eq, 1).astype(jnp.float32)
    out_ref[...] = jnp.broadcast_to(total / jnp.float32(_K), (1, 1))


def kernel(y_hat, y):
    yht = y_hat.T
    yt = y.T
    nll, tl = pl.pallas_call(
        _phase1_body,
        grid=(_NB,),
        in_specs=[
            pl.BlockSpec((_C, _BN), lambda i: (0, i)),
            pl.BlockSpec((_C, _BN), lambda i: (0, i)),
        ],
        out_specs=[
            pl.BlockSpec((1, _BN), lambda i: (0, i)),
            pl.BlockSpec((1, _BN), lambda i: (0, i)),
        ],
        out_shape=[
            jax.ShapeDtypeStruct((1, _B), jnp.float32),
            jax.ShapeDtypeStruct((1, _B), jnp.float32),
        ],
    )(yht, yt)

    nll2 = nll.reshape(128, 128)
    tl2 = tl.reshape(128, 128)
    out = pl.pallas_call(
        _phase2_body,
        out_shape=jax.ShapeDtypeStruct((1, 1), jnp.float32),
    )(nll2, tl2)
    return out[0, 0]
